# baseline (speedup 1.0000x reference)
import functools

import jax
import jax.numpy as jnp
from jax import lax
from jax.experimental import pallas as pl
from jax.experimental.pallas import tpu as pltpu
from jax.experimental.pallas import tpu_sc as plsc

_IGNORE = -100
_B, _S, _C = 16, 2048, 2048
_BS = 2048
_NSUB = 32
_TOK = (_B * _S) // _NSUB
_HALF = _S // 2


_LOG2E = 1.4426950408889634
_LN2 = 0.6931471805599453


def _stream_body(logits_ref, labels_ref, preds_ref, perb_ref):
    x = logits_ref[0]
    lab = labels_ref[0]
    mask = lab != _IGNORE
    lab_safe = jnp.where(mask, lab, 0)
    m = jnp.max(x, axis=1, keepdims=True)
    iota = lax.broadcasted_iota(jnp.int32, x.shape, 1)
    amax = jnp.min(jnp.where(x == m, iota, _C), axis=1, keepdims=True)
    d = m * (-_LOG2E)
    se = jnp.sum(jnp.exp2(x * _LOG2E + d), axis=1, keepdims=True)
    lse = m + jnp.log(se)
    ll = jnp.sum(jnp.where(iota == lab_safe, x, 0.0), axis=1, keepdims=True)
    preds_ref[0] = amax
    maskf = mask.astype(jnp.float32)
    tl = jnp.where(mask, lse - ll, 0.0)
    corrf = (mask & (amax == lab_safe)).astype(jnp.float32)
    perb_ref[0, 0, 0] = jnp.sum(maskf)
    perb_ref[0, 0, 1] = jnp.sum(corrf)
    perb_ref[0, 0, 2] = jnp.sum(tl)


def _stream_pass(logits, labels3):
    grid = (_B,)
    return pl.pallas_call(
        _stream_body,
        grid=grid,
        compiler_params=pltpu.CompilerParams(
            vmem_limit_bytes=120 * 1024 * 1024,
        ),
        in_specs=[
            pl.BlockSpec((1, _BS, _C), lambda b: (b, 0, 0)),
            pl.BlockSpec((1, _BS, 1), lambda b: (b, 0, 0)),
        ],
        out_specs=[
            pl.BlockSpec((1, _BS, 1), lambda b: (b, 0, 0)),
            pl.BlockSpec((1, 1, 4), lambda b: (b, 0, 0), memory_space=pltpu.SMEM),
        ],
        out_shape=[
            jax.ShapeDtypeStruct((_B, _S, 1), jnp.int32),
            jax.ShapeDtypeStruct((_B, 1, 4), jnp.float32),
        ],
    )(logits, labels3)


def _hist_body(lab_hbm, prd_hbm, out_hbm, labv, prdv, tpv, fpv, fnv):
    wid = lax.axis_index("s") * 2 + lax.axis_index("c")
    base = wid * _TOK
    pltpu.sync_copy(lab_hbm.at[pl.ds(base, _TOK)], labv)
    pltpu.sync_copy(prd_hbm.at[pl.ds(base, _TOK)], prdv)

    zeros = jnp.zeros((16,), jnp.float32)

    def zbody(j, carry):
        tpv[pl.ds(j * 16, 16)] = zeros
        fpv[pl.ds(j * 16, 16)] = zeros
        fnv[pl.ds(j * 16, 16)] = zeros
        return carry

    lax.fori_loop(0, _C // 16, zbody, 0)

    def body(i, carry):
        l16 = labv[pl.ds(i * 16, 16)]
        p16 = prdv[pl.ds(i * 16, 16)]
        mask = l16 != _IGNORE
        ls = jnp.where(mask, l16, 0)
        wm = jnp.where(mask, 1.0, 0.0).astype(jnp.float32)
        corr = mask & (p16 == l16)
        wc = jnp.where(corr, 1.0, 0.0).astype(jnp.float32)
        ww = wm - wc
        plsc.addupdate_scatter(tpv, [ls], wc)
        plsc.addupdate_scatter(fnv, [ls], ww)
        plsc.addupdate_scatter(fpv, [p16], ww)
        return carry

    lax.fori_loop(0, _TOK // 16, body, 0)

    obase = wid * 3 * _C
    pltpu.sync_copy(tpv, out_hbm.at[pl.ds(obase, _C)])
    pltpu.sync_copy(fpv, out_hbm.at[pl.ds(obase + _C, _C)])
    pltpu.sync_copy(fnv, out_hbm.at[pl.ds(obase + 2 * _C, _C)])


@functools.cache
def _hist_pass():
    return pl.kernel(
        _hist_body,
        out_type=jax.ShapeDtypeStruct((_NSUB * 3 * _C,), jnp.float32),
        mesh=plsc.VectorSubcoreMesh(
            core_axis_name="c", subcore_axis_name="s", num_cores=2, num_subcores=16
        ),
        compiler_params=pltpu.CompilerParams(needs_layout_passes=False),
        scratch_types=[
            pltpu.VMEM((_TOK,), jnp.int32),
            pltpu.VMEM((_TOK,), jnp.int32),
            pltpu.VMEM((_C,), jnp.float32),
            pltpu.VMEM((_C,), jnp.float32),
            pltpu.VMEM((_C,), jnp.float32),
        ],
    )


def _final_body(perb_ref, hist_ref, qh_ref, hal_ref, st_ref, out_ref):
    perb = perb_ref[...].reshape(_B, 4)
    lc = perb[:, 0:1]
    nc = perb[:, 1:2]
    tls = perb[:, 2:3]
    seqc = nc == lc
    seqcf = seqc.astype(jnp.float32)
    valid = (hal_ref[...] != 0) & (lc > 0.0)
    validf = valid.astype(jnp.float32)
    div = jnp.maximum(lc, 1.0)
    acc = validf * (nc / div)
    lm_loss = jnp.sum(tls / div)

    hist = hist_ref[...]
    hsum = hist[:, 0] + hist[:, 1]
    tp = hsum[:, 0]
    fp = hsum[:, 1]
    fn = hsum[:, 2]
    denom_p = tp + fp
    precision = jnp.where(denom_p > 0, tp / jnp.maximum(denom_p, 1e-9), 0.0)
    denom_r = tp + fn
    recall = jnp.where(denom_r > 0, tp / jnp.maximum(denom_r, 1e-9), 0.0)
    denom_f = precision + recall
    f1c = jnp.where(
        denom_f > 0, 2.0 * precision * recall / jnp.maximum(denom_f, 1e-9), 0.0
    )
    sm = ((tp + fp + fn) > 0).astype(jnp.float32)
    supp = jnp.maximum(jnp.sum(sm, axis=1, keepdims=True), 1.0)
    f1seq = validf * (jnp.sum(f1c * sm, axis=1, keepdims=True) / supp)

    x = qh_ref[...]
    q_halt_loss = jnp.sum(
        jnp.maximum(x, 0.0) - x * seqcf + jnp.log(1.0 + jnp.exp(-jnp.abs(x)))
    )
    qacc = jnp.sum(validf * ((x >= 0.0) == seqc).astype(jnp.float32))

    out_ref[0] = lm_loss + 0.5 * q_halt_loss
    out_ref[1] = lm_loss
    out_ref[2] = q_halt_loss
    out_ref[3] = jnp.sum(validf)
    out_ref[4] = jnp.sum(acc)
    out_ref[5] = jnp.sum(acc * acc)
    out_ref[6] = jnp.sum(f1seq)
    out_ref[7] = jnp.sum(validf * seqcf)
    out_ref[8] = qacc
    out_ref[9] = jnp.sum(validf * st_ref[...])


def _final_pass(perb, hists, qh, halted_i, steps_f):
    return pl.pallas_call(
        _final_body,
        in_specs=[pl.BlockSpec(memory_space=pltpu.VMEM)] * 5,
        out_specs=pl.BlockSpec(memory_space=pltpu.SMEM),
        out_shape=jax.ShapeDtypeStruct((16,), jnp.float32),
    )(perb, hists, qh, halted_i, steps_f)


def kernel(logits, labels, q_halt_logits, halted, steps):
    labels_flat = labels.reshape(-1)
    preds3, perb = _stream_pass(logits, labels.reshape(_B, _S, 1))
    preds = preds3.reshape(_B, _S)
    hists = _hist_pass()(labels_flat, preds.reshape(-1))
    hists = hists.reshape(_B, 2, 3, _C)
    out = _final_pass(
        perb,
        hists,
        q_halt_logits.reshape(_B, 1),
        halted.reshape(_B, 1).astype(jnp.int32),
        steps.reshape(_B, 1).astype(jnp.float32),
    )
    total_loss = out[0]
    lm_loss = out[1]
    q_halt_loss = out[2]
    count = out[3].astype(jnp.int32)
    accuracy = out[4]
    accuracy_sq = out[5]
    f1 = out[6]
    exact_accuracy = out[7].astype(jnp.int32)
    q_halt_accuracy = out[8].astype(jnp.int32)
    steps_sum = out[9].astype(jnp.int32)
    return (
        total_loss,
        lm_loss,
        q_halt_loss,
        preds,
        count,
        accuracy,
        accuracy_sq,
        f1,
        exact_accuracy,
        q_halt_accuracy,
        steps_sum,
    )


# speedup vs baseline: 1.0010x; 1.0010x over previous
import functools

import jax
import jax.numpy as jnp
from jax import lax
from jax.experimental import pallas as pl
from jax.experimental.pallas import tpu as pltpu
from jax.experimental.pallas import tpu_sc as plsc

_IGNORE = -100
_B, _S, _C = 16, 2048, 2048
_BS = 2048
_NSUB = 32
_TOK = (_B * _S) // _NSUB
_LOG2E = 1.4426950408889634


def _stream_body(logits_ref, labels_ref, preds_ref, perb_ref):
    x = logits_ref[0]
    lab = labels_ref[0]
    mask = lab != _IGNORE
    lab_safe = jnp.where(mask, lab, 0)
    m = jnp.max(x, axis=1, keepdims=True)
    iota = lax.broadcasted_iota(jnp.int32, x.shape, 1)
    amax = jnp.min(jnp.where(x == m, iota, _C), axis=1, keepdims=True)
    d = m * (-_LOG2E)
    se = jnp.sum(jnp.exp2(x * _LOG2E + d), axis=1, keepdims=True)
    lse = m + jnp.log(se)
    ll = jnp.sum(jnp.where(iota == lab_safe, x, 0.0), axis=1, keepdims=True)
    preds_ref[0] = amax
    maskf = mask.astype(jnp.float32)
    tl = jnp.where(mask, lse - ll, 0.0)
    corrf = (mask & (amax == lab_safe)).astype(jnp.float32)
    perb_ref[0, 0, 0] = jnp.sum(maskf)
    perb_ref[0, 0, 1] = jnp.sum(corrf)
    perb_ref[0, 0, 2] = jnp.sum(tl)


def _stream_pass(logits, labels3):
    grid = (_B,)
    return pl.pallas_call(
        _stream_body,
        grid=grid,
        in_specs=[
            pl.BlockSpec((1, _BS, _C), lambda b: (b, 0, 0)),
            pl.BlockSpec((1, _BS, 1), lambda b: (b, 0, 0)),
        ],
        out_specs=[
            pl.BlockSpec((1, _BS, 1), lambda b: (b, 0, 0)),
            pl.BlockSpec((1, 1, 4), lambda b: (b, 0, 0), memory_space=pltpu.SMEM),
        ],
        out_shape=[
            jax.ShapeDtypeStruct((_B, _S, 1), jnp.int32),
            jax.ShapeDtypeStruct((_B, 1, 4), jnp.float32),
        ],
    )(logits, labels3)


def _hist_body(lab_hbm, prd_hbm, out_hbm, labv, prdv, tpv, fpv, fnv):
    wid = lax.axis_index("s") * 2 + lax.axis_index("c")
    base = wid * _TOK
    pltpu.sync_copy(lab_hbm.at[pl.ds(base, _TOK)], labv)
    pltpu.sync_copy(prd_hbm.at[pl.ds(base, _TOK)], prdv)

    zeros = jnp.zeros((16,), jnp.float32)

    def zbody(j, carry):
        tpv[pl.ds(j * 16, 16)] = zeros
        fpv[pl.ds(j * 16, 16)] = zeros
        fnv[pl.ds(j * 16, 16)] = zeros
        return carry

    lax.fori_loop(0, _C // 16, zbody, 0)

    def body(i, carry):
        l16 = labv[pl.ds(i * 16, 16)]
        p16 = prdv[pl.ds(i * 16, 16)]
        mask = l16 != _IGNORE
        ls = jnp.where(mask, l16, 0)
        wm = jnp.where(mask, 1.0, 0.0).astype(jnp.float32)
        corr = mask & (p16 == l16)
        wc = jnp.where(corr, 1.0, 0.0).astype(jnp.float32)
        ww = wm - wc
        plsc.addupdate_scatter(tpv, [ls], wc)
        plsc.addupdate_scatter(fnv, [ls], ww)
        plsc.addupdate_scatter(fpv, [p16], ww)
        return carry

    lax.fori_loop(0, _TOK // 16, body, 0)

    obase = wid * 3 * _C
    pltpu.sync_copy(tpv, out_hbm.at[pl.ds(obase, _C)])
    pltpu.sync_copy(fpv, out_hbm.at[pl.ds(obase + _C, _C)])
    pltpu.sync_copy(fnv, out_hbm.at[pl.ds(obase + 2 * _C, _C)])


@functools.cache
def _hist_pass():
    return pl.kernel(
        _hist_body,
        out_type=jax.ShapeDtypeStruct((_NSUB * 3 * _C,), jnp.float32),
        mesh=plsc.VectorSubcoreMesh(
            core_axis_name="c", subcore_axis_name="s", num_cores=2, num_subcores=16
        ),
        compiler_params=pltpu.CompilerParams(needs_layout_passes=False),
        scratch_types=[
            pltpu.VMEM((_TOK,), jnp.int32),
            pltpu.VMEM((_TOK,), jnp.int32),
            pltpu.VMEM((_C,), jnp.float32),
            pltpu.VMEM((_C,), jnp.float32),
            pltpu.VMEM((_C,), jnp.float32),
        ],
    )


def _final_body(perb_ref, hist_ref, qh_ref, hal_ref, st_ref, out_ref):
    perb = perb_ref[...].reshape(_B, 4)
    lc = perb[:, 0:1]
    nc = perb[:, 1:2]
    tls = perb[:, 2:3]
    seqc = nc == lc
    seqcf = seqc.astype(jnp.float32)
    valid = (hal_ref[...] != 0) & (lc > 0.0)
    validf = valid.astype(jnp.float32)
    div = jnp.maximum(lc, 1.0)
    acc = validf * (nc / div)
    lm_loss = jnp.sum(tls / div)

    hist = hist_ref[...]
    hsum = hist[:, 0] + hist[:, 1]
    tp = hsum[:, 0]
    fp = hsum[:, 1]
    fn = hsum[:, 2]
    denom_p = tp + fp
    precision = jnp.where(denom_p > 0, tp / jnp.maximum(denom_p, 1e-9), 0.0)
    denom_r = tp + fn
    recall = jnp.where(denom_r > 0, tp / jnp.maximum(denom_r, 1e-9), 0.0)
    denom_f = precision + recall
    f1c = jnp.where(
        denom_f > 0, 2.0 * precision * recall / jnp.maximum(denom_f, 1e-9), 0.0
    )
    sm = ((tp + fp + fn) > 0).astype(jnp.float32)
    supp = jnp.maximum(jnp.sum(sm, axis=1, keepdims=True), 1.0)
    f1seq = validf * (jnp.sum(f1c * sm, axis=1, keepdims=True) / supp)

    x = qh_ref[...]
    q_halt_loss = jnp.sum(
        jnp.maximum(x, 0.0) - x * seqcf + jnp.log(1.0 + jnp.exp(-jnp.abs(x)))
    )
    qacc = jnp.sum(validf * ((x >= 0.0) == seqc).astype(jnp.float32))

    out_ref[0] = lm_loss + 0.5 * q_halt_loss
    out_ref[1] = lm_loss
    out_ref[2] = q_halt_loss
    out_ref[3] = jnp.sum(validf)
    out_ref[4] = jnp.sum(acc)
    out_ref[5] = jnp.sum(acc * acc)
    out_ref[6] = jnp.sum(f1seq)
    out_ref[7] = jnp.sum(validf * seqcf)
    out_ref[8] = qacc
    out_ref[9] = jnp.sum(validf * st_ref[...])


def _final_pass(perb, hists, qh, halted_i, steps_f):
    return pl.pallas_call(
        _final_body,
        in_specs=[pl.BlockSpec(memory_space=pltpu.VMEM)] * 5,
        out_specs=pl.BlockSpec(memory_space=pltpu.SMEM),
        out_shape=jax.ShapeDtypeStruct((16,), jnp.float32),
    )(perb, hists, qh, halted_i, steps_f)


def kernel(logits, labels, q_halt_logits, halted, steps):
    labels_flat = labels.reshape(-1)
    preds3, perb = _stream_pass(logits, labels.reshape(_B, _S, 1))
    preds = preds3.reshape(_B, _S)
    hists = _hist_pass()(labels_flat, preds.reshape(-1))
    hists = hists.reshape(_B, 2, 3, _C)
    out = _final_pass(
        perb,
        hists,
        q_halt_logits.reshape(_B, 1),
        halted.reshape(_B, 1).astype(jnp.int32),
        steps.reshape(_B, 1).astype(jnp.float32),
    )
    total_loss = out[0]
    lm_loss = out[1]
    q_halt_loss = out[2]
    count = out[3].astype(jnp.int32)
    accuracy = out[4]
    accuracy_sq = out[5]
    f1 = out[6]
    exact_accuracy = out[7].astype(jnp.int32)
    q_halt_accuracy = out[8].astype(jnp.int32)
    steps_sum = out[9].astype(jnp.int32)
    return (
        total_loss,
        lm_loss,
        q_halt_loss,
        preds,
        count,
        accuracy,
        accuracy_sq,
        f1,
        exact_accuracy,
        q_halt_accuracy,
        steps_sum,
    )
